# baseline (speedup 1.0000x reference)
import functools

import jax
import jax.numpy as jnp
import numpy as np
from jax import lax
from jax.experimental import pallas as pl
from jax.experimental.pallas import tpu as pltpu
from jax.experimental.pallas import tpu_sc as plsc
from jax.experimental import layout as jex_layout

B, C, A, D = 1024, 50, 4, 64
VOCAB0, VOCAB1 = 100000, 1000
CA = C * A
PK = 224

NC, NS = 2, 16
NW = NC * NS
BPW = B // NW


def _pos_encoding_np():
    pos = np.arange(C)[:, np.newaxis].astype(np.float32)
    i = np.arange(D)[np.newaxis, :].astype(np.float32)
    angle = pos / np.power(10000, 2.0 * (np.floor(i / 2.0)) / np.float32(D))
    angle[:, 0::2] = np.sin(angle[:, 0::2])
    angle[:, 1::2] = np.cos(angle[:, 1::2])
    return angle


_POS = _pos_encoding_np()


def _sc_body(pk_hbm, ta_hbm, tr_hbm, pos_hbm, out_hbm,
             pk0_v, pk1_v, pk2_v, pk3_v, st0a_v, st0b_v, st1a_v, st1b_v,
             bufa_v, bufb_v, pos_v, sem_i, sem_g, sem_o):
    wid = lax.axis_index("s") * NC + lax.axis_index("c")
    pltpu.sync_copy(pos_hbm, pos_v)

    pk_slots = [pk0_v, pk1_v, pk2_v, pk3_v]
    st0_slots = [st0a_v, st0b_v]
    st1_slots = [st1a_v, st1b_v]
    buf_slots = [bufa_v, bufb_v]
    b00 = wid * BPW

    def fire_in(k):
        return pltpu.async_copy(
            pk_hbm.at[pl.ds(b00 + k, 1)], pk_slots[k % 4], sem_i)

    def fire_gathers(k):
        p = k % 2
        pk = pk_slots[k % 4]
        g0 = pltpu.async_copy(
            ta_hbm.at[pk.at[0, pl.ds(0, 56)]], st0_slots[p], sem_g)
        g1 = pltpu.async_copy(
            tr_hbm.at[pk.at[0, pl.ds(56, 56)]], st1_slots[p], sem_g)
        return g0, g1

    def assemble(k):
        p = k % 2
        pk = pk_slots[k % 4]
        st0, st1, buf = st0_slots[p], st1_slots[p], buf_slots[p]

        def ev_body(c, carry2):
            zsp = jnp.full((16,), 0, jnp.int32)
            csp = jnp.full((16,), 112, jnp.int32) + c
            n0 = plsc.bitcast(plsc.load_gather(pk, [zsp, csp]), jnp.float32)
            n1 = plsc.bitcast(plsc.load_gather(pk, [zsp, csp + 56]),
                              jnp.float32)
            for j in range(D // 16):
                pvec = pos_v[c, pl.ds(j * 16, 16)]
                v0 = st0[c, pl.ds(j * 16, 16)]
                v1 = st1[c, pl.ds(j * 16, 16)]
                buf[0, c * A, pl.ds(j * 16, 16)] = v0 + pvec
                buf[0, c * A + 1, pl.ds(j * 16, 16)] = v1 + pvec
                buf[0, c * A + 2, pl.ds(j * 16, 16)] = n0 + pvec
                buf[0, c * A + 3, pl.ds(j * 16, 16)] = n1 + pvec
            return carry2

        lax.fori_loop(0, C, ev_body, 0)

    def fire_out(k):
        return pltpu.async_copy(
            buf_slots[k % 2], out_hbm.at[pl.ds(b00 + k, 1)], sem_o)

    in_cp = {0: fire_in(0), 1: fire_in(1)}
    g_cp = {}
    out_cp = {}
    for k in range(BPW):
        in_cp[k].wait()
        g_cp[k] = fire_gathers(k)
        if k + 2 < BPW:
            in_cp[k + 2] = fire_in(k + 2)
        if k >= 1:
            g_cp[k - 1][0].wait()
            g_cp[k - 1][1].wait()
            if k >= 3:
                out_cp[k - 3].wait()
            assemble(k - 1)
            out_cp[k - 1] = fire_out(k - 1)
    g_cp[BPW - 1][0].wait()
    g_cp[BPW - 1][1].wait()
    out_cp[BPW - 3].wait()
    assemble(BPW - 1)
    out_cp[BPW - 1] = fire_out(BPW - 1)
    out_cp[BPW - 2].wait()
    out_cp[BPW - 1].wait()


def kernel(inputs, table_activity, table_resource):
    pos = jnp.asarray(_POS)
    idx0 = inputs[:, 0::4].astype(jnp.int32)
    idx1 = inputs[:, 1::4].astype(jnp.int32)
    n0b = jax.lax.bitcast_convert_type(inputs[:, 2::4], jnp.int32)
    n1b = jax.lax.bitcast_convert_type(inputs[:, 3::4], jnp.int32)
    z6 = jnp.zeros((B, 6), jnp.int32)
    packed = jnp.concatenate([idx0, z6, idx1, z6, n0b, z6, n1b, z6], axis=1)
    ta128 = jnp.pad(table_activity, ((0, 0), (0, 128 - D)))
    tr128 = jnp.pad(table_resource, ((0, 0), (0, 128 - D)))
    mesh = plsc.VectorSubcoreMesh(core_axis_name="c", subcore_axis_name="s")
    k = functools.partial(
        pl.kernel,
        out_type=jax.ShapeDtypeStruct((B, CA, D), jnp.float32),
        mesh=mesh,
        compiler_params=pltpu.CompilerParams(use_tc_tiling_on_sc=True,
                                             needs_layout_passes=False),
        scratch_types=(
            [pltpu.VMEM((1, PK), jnp.int32)] * 4 +
            [pltpu.VMEM((56, 128), jnp.float32)] * 4 +
            [pltpu.VMEM((1, CA, D), jnp.float32)] * 2 +
            [pltpu.VMEM((C, D), jnp.float32),
             pltpu.SemaphoreType.DMA,
             pltpu.SemaphoreType.DMA,
             pltpu.SemaphoreType.DMA]
        ),
    )(_sc_body)
    out = k(packed, ta128, tr128, pos)
    return jex_layout.with_layout_constraint(
        out, jex_layout.Layout(major_to_minor=(0, 1, 2)))


# speedup vs baseline: 1.0077x; 1.0077x over previous
import functools

import jax
import jax.numpy as jnp
import numpy as np
from jax import lax
from jax.experimental import pallas as pl
from jax.experimental.pallas import tpu as pltpu
from jax.experimental.pallas import tpu_sc as plsc
from jax.experimental import layout as jex_layout

B, C, A, D = 1024, 50, 4, 64
VOCAB0, VOCAB1 = 100000, 1000
CA = C * A
PK = 224
GL = 56

NC, NS = 2, 16
NW = NC * NS
BPW = B // NW
NB = 2
NCHUNK = BPW // NB


def _pos_encoding_np():
    pos = np.arange(C)[:, np.newaxis].astype(np.float32)
    i = np.arange(D)[np.newaxis, :].astype(np.float32)
    angle = pos / np.power(10000, 2.0 * (np.floor(i / 2.0)) / np.float32(D))
    angle[:, 0::2] = np.sin(angle[:, 0::2])
    angle[:, 1::2] = np.cos(angle[:, 1::2])
    return angle


_POS = _pos_encoding_np()


def _sc_body(pk_hbm, ta_hbm, tr_hbm, pos_hbm, out_hbm,
             pka_v, pkb_v, st0a_v, st0b_v, st1a_v, st1b_v, buf_v, pos_v,
             sem_ga, sem_gb, sem_o):
    wid = lax.axis_index("s") * NC + lax.axis_index("c")
    pltpu.sync_copy(pos_hbm, pos_v)

    pk_slots = [pka_v, pkb_v]
    st0_slots = [st0a_v, st0b_v]
    st1_slots = [st1a_v, st1b_v]
    g_sems = [sem_ga, sem_gb]
    b00 = wid * BPW

    def stage(k, p):
        pk = pk_slots[p]
        pltpu.sync_copy(pk_hbm.at[pl.ds(b00 + k * NB, NB)], pk)
        for b in range(NB):
            pltpu.async_copy(ta_hbm.at[pk.at[b, pl.ds(0, GL)]],
                             st0_slots[p].at[pl.ds(b * GL, GL)], g_sems[p])
            pltpu.async_copy(tr_hbm.at[pk.at[b, pl.ds(GL, GL)]],
                             st1_slots[p].at[pl.ds(b * GL, GL)], g_sems[p])

    def drain_gathers(p):
        for _ in range(2 * NB):
            pltpu.make_async_copy(
                ta_hbm.at[pl.ds(0, GL)],
                st0_slots[p].at[pl.ds(0, GL)], g_sems[p]).wait()

    def drain_out():
        pltpu.make_async_copy(
            buf_v, out_hbm.at[pl.ds(0, NB)], sem_o).wait()

    stage(0, 0)

    def do_chunk(k, p):
        @pl.when(k + 1 < NCHUNK)
        def _():
            stage(k + 1, 1 - p)

        drain_gathers(p)

        @pl.when(k >= 1)
        def _():
            drain_out()

        pk = pk_slots[p]
        st0, st1 = st0_slots[p], st1_slots[p]

        def ev_body(c, carry2):
            for b in range(NB):
                e = b * GL + c
                zsp = jnp.full((16,), b, jnp.int32)
                csp = jnp.full((16,), 112, jnp.int32) + c
                n0 = plsc.bitcast(plsc.load_gather(pk, [zsp, csp]),
                                  jnp.float32)
                n1 = plsc.bitcast(plsc.load_gather(pk, [zsp, csp + GL]),
                                  jnp.float32)
                for j in range(D // 16):
                    pvec = pos_v[c, pl.ds(j * 16, 16)]
                    v0 = st0[e, pl.ds(j * 16, 16)]
                    v1 = st1[e, pl.ds(j * 16, 16)]
                    buf_v[b, c * A, pl.ds(j * 16, 16)] = v0 + pvec
                    buf_v[b, c * A + 1, pl.ds(j * 16, 16)] = v1 + pvec
                    buf_v[b, c * A + 2, pl.ds(j * 16, 16)] = n0 + pvec
                    buf_v[b, c * A + 3, pl.ds(j * 16, 16)] = n1 + pvec
            return carry2

        lax.fori_loop(0, C, ev_body, 0)

        pltpu.async_copy(buf_v, out_hbm.at[pl.ds(b00 + k * NB, NB)], sem_o)

    def chunk_pair(i, carry):
        do_chunk(2 * i, 0)
        do_chunk(2 * i + 1, 1)
        return carry

    lax.fori_loop(0, NCHUNK // 2, chunk_pair, 0)
    drain_out()


def kernel(inputs, table_activity, table_resource):
    pos = jnp.asarray(_POS)
    idx0 = inputs[:, 0::4].astype(jnp.int32)
    idx1 = inputs[:, 1::4].astype(jnp.int32)
    n0b = jax.lax.bitcast_convert_type(inputs[:, 2::4], jnp.int32)
    n1b = jax.lax.bitcast_convert_type(inputs[:, 3::4], jnp.int32)
    z6 = jnp.zeros((B, 6), jnp.int32)
    packed = jnp.concatenate([idx0, z6, idx1, z6, n0b, z6, n1b, z6], axis=1)
    ta128 = jnp.pad(table_activity, ((0, 0), (0, 128 - D)))
    tr128 = jnp.pad(table_resource, ((0, 0), (0, 128 - D)))
    mesh = plsc.VectorSubcoreMesh(core_axis_name="c", subcore_axis_name="s")
    k = functools.partial(
        pl.kernel,
        out_type=jax.ShapeDtypeStruct((B, CA, D), jnp.float32),
        mesh=mesh,
        compiler_params=pltpu.CompilerParams(use_tc_tiling_on_sc=True,
                                             needs_layout_passes=False),
        scratch_types=(
            [pltpu.VMEM((NB, PK), jnp.int32)] * 2 +
            [pltpu.VMEM((NB * GL, 128), jnp.float32)] * 4 +
            [pltpu.VMEM((NB, CA, D), jnp.float32),
             pltpu.VMEM((C, D), jnp.float32),
             pltpu.SemaphoreType.DMA,
             pltpu.SemaphoreType.DMA,
             pltpu.SemaphoreType.DMA]
        ),
    )(_sc_body)
    out = k(packed, ta128, tr128, pos)
    return jex_layout.with_layout_constraint(
        out, jex_layout.Layout(major_to_minor=(0, 1, 2)))
